# baseline (speedup 1.0000x reference)
import functools

import jax
import jax.numpy as jnp
from jax import lax
from jax.experimental import pallas as pl
from jax.experimental.pallas import tpu as pltpu
from jax.experimental.pallas import tpu_sc as plsc

INPUT_DIM = 1000000
EMBED_DIM = 64
OUTPUT_DIM = 128
BATCH = 16384

_NC = 2
_NS = 16
_NW = _NC * _NS
_B_PER_W = BATCH // _NW
_CHUNK = 128
_N_CHUNKS = _B_PER_W // _CHUNK

_sc_mesh = plsc.VectorSubcoreMesh(core_axis_name="c", subcore_axis_name="s")


@functools.partial(
    pl.kernel,
    mesh=_sc_mesh,
    compiler_params=pltpu.CompilerParams(use_tc_tiling_on_sc=False),
    out_type=jax.ShapeDtypeStruct((BATCH, EMBED_DIM), jnp.float32),
    scratch_types=[
        pltpu.VMEM((_B_PER_W,), jnp.int32),
        pltpu.VMEM((_B_PER_W, EMBED_DIM), jnp.float32),
        pltpu.SemaphoreType.DMA,
    ],
)
def _sc_gather(table_hbm, idx_hbm, out_hbm, idx_v, rows_v, sem):
    wid = lax.axis_index("s") * _NC + lax.axis_index("c")
    base = wid * _B_PER_W
    pltpu.sync_copy(idx_hbm.at[pl.ds(base, _B_PER_W)], idx_v)
    copies = []
    for j in range(_N_CHUNKS):
        copies.append(
            pltpu.async_copy(
                table_hbm.at[idx_v.at[pl.ds(j * _CHUNK, _CHUNK)]],
                rows_v.at[pl.ds(j * _CHUNK, _CHUNK)],
                sem,
            )
        )
    for c in copies:
        c.wait()
    pltpu.sync_copy(rows_v, out_hbm.at[pl.ds(base, _B_PER_W)])


def _mm_body(e_ref, w_ref, b_ref, o_ref):
    o_ref[...] = (
        lax.dot_general(
            e_ref[...], w_ref[...],
            (((1,), (1,)), ((), ())),
            preferred_element_type=jnp.float32,
        )
        + b_ref[...]
    )


_M_BLK = 2048


def _tc_linear(e, fc_w, fc_b2):
    return pl.pallas_call(
        _mm_body,
        grid=(BATCH // _M_BLK,),
        in_specs=[
            pl.BlockSpec((_M_BLK, EMBED_DIM), lambda i: (i, 0)),
            pl.BlockSpec((OUTPUT_DIM, EMBED_DIM), lambda i: (0, 0)),
            pl.BlockSpec((1, OUTPUT_DIM), lambda i: (0, 0)),
        ],
        out_specs=pl.BlockSpec((_M_BLK, OUTPUT_DIM), lambda i: (i, 0)),
        out_shape=jax.ShapeDtypeStruct((BATCH, OUTPUT_DIM), jnp.float32),
    )(e, fc_w, fc_b2)


def kernel(_x, x, emb_table, fc_w, fc_b):
    idx = x.astype(jnp.int32)
    e = _sc_gather(emb_table, idx)
    return _tc_linear(e, fc_w, fc_b.reshape(1, OUTPUT_DIM))


# speedup vs baseline: 1.7173x; 1.7173x over previous
import functools

import jax
import jax.numpy as jnp
from jax import lax
from jax.experimental import pallas as pl
from jax.experimental.pallas import tpu as pltpu
from jax.experimental.pallas import tpu_sc as plsc

INPUT_DIM = 1000000
EMBED_DIM = 64
OUTPUT_DIM = 128
BATCH = 16384

_NC = 2
_NS = 16
_NW = _NC * _NS
_B_PER_W = BATCH // _NW

_sc_mesh = plsc.VectorSubcoreMesh(core_axis_name="c", subcore_axis_name="s")


@functools.partial(
    pl.kernel,
    mesh=_sc_mesh,
    out_type=jax.ShapeDtypeStruct((BATCH, EMBED_DIM), jnp.float32),
    scratch_types=[
        pltpu.SMEM((_B_PER_W,), jnp.int32),
        pltpu.VMEM((_B_PER_W,), jnp.int32),
        pltpu.VMEM((_B_PER_W, EMBED_DIM), jnp.float32),
        pltpu.SemaphoreType.DMA,
    ],
)
def _sc_gather(table_hbm, idx_hbm, out_hbm, idx_s, idx_v, rows_v, sem):
    wid = lax.axis_index("s") * _NC + lax.axis_index("c")
    base = wid * _B_PER_W
    pltpu.sync_copy(idx_hbm.at[pl.ds(base, _B_PER_W)], idx_v)

    @pl.loop(0, _B_PER_W, step=16)
    def _grp(g):
        vec = idx_v[pl.ds(g, 16)]
        for l in range(16):
            r = vec[l]
            pltpu.async_copy(
                table_hbm.at[pl.ds(r, 1)], rows_v.at[pl.ds(g + l, 1)], sem
            )
    pltpu.make_async_copy(
        table_hbm.at[pl.ds(0, _B_PER_W)], rows_v, sem
    ).wait()
    pltpu.sync_copy(rows_v, out_hbm.at[pl.ds(base, _B_PER_W)])


def _mm_body(e_ref, w_ref, b_ref, o_ref):
    o_ref[...] = (
        lax.dot_general(
            e_ref[...], w_ref[...],
            (((1,), (1,)), ((), ())),
            preferred_element_type=jnp.float32,
        )
        + b_ref[...]
    )


_M_BLK = 2048


def _tc_linear(e, fc_w, fc_b2):
    return pl.pallas_call(
        _mm_body,
        grid=(BATCH // _M_BLK,),
        in_specs=[
            pl.BlockSpec((_M_BLK, EMBED_DIM), lambda i: (i, 0)),
            pl.BlockSpec((OUTPUT_DIM, EMBED_DIM), lambda i: (0, 0)),
            pl.BlockSpec((1, OUTPUT_DIM), lambda i: (0, 0)),
        ],
        out_specs=pl.BlockSpec((_M_BLK, OUTPUT_DIM), lambda i: (i, 0)),
        out_shape=jax.ShapeDtypeStruct((BATCH, OUTPUT_DIM), jnp.float32),
    )(e, fc_w, fc_b2)


def kernel(_x, x, emb_table, fc_w, fc_b):
    idx = x.astype(jnp.int32)
    e = _sc_gather(emb_table, idx)
    return _tc_linear(e, fc_w, fc_b.reshape(1, OUTPUT_DIM))


# speedup vs baseline: 2.3167x; 1.3491x over previous
import functools

import jax
import jax.numpy as jnp
from jax import lax
from jax.experimental import pallas as pl
from jax.experimental.pallas import tpu as pltpu
from jax.experimental.pallas import tpu_sc as plsc

INPUT_DIM = 1000000
EMBED_DIM = 64
OUTPUT_DIM = 128
BATCH = 16384

_NC = 2
_NS = 16
_NW = _NC * _NS
_B_PER_W = BATCH // _NW
_LANES = 128
_NBUF = 4

_sc_mesh = plsc.VectorSubcoreMesh(core_axis_name="c", subcore_axis_name="s")


@functools.partial(
    pl.kernel,
    mesh=_sc_mesh,
    compiler_params=pltpu.CompilerParams(needs_layout_passes=False),
    out_type=jax.ShapeDtypeStruct((EMBED_DIM, BATCH), jnp.float32),
    scratch_types=[
        pltpu.VMEM((_B_PER_W,), jnp.int32),
        pltpu.VMEM((EMBED_DIM, _B_PER_W), jnp.float32),
        [pltpu.VMEM((EMBED_DIM, _LANES), jnp.float32) for _ in range(_NBUF)],
        [pltpu.SemaphoreType.DMA for _ in range(_NBUF)],
    ],
)
def _sc_gather_t(table_t_hbm, idx_hbm, out_hbm, idx_v, cols_v, stages, sems):
    wid = lax.axis_index("s") * _NC + lax.axis_index("c")
    base = wid * _B_PER_W
    pltpu.sync_copy(idx_hbm.at[pl.ds(base, _B_PER_W)], idx_v)

    iota16 = lax.iota(jnp.int32, 16)

    @pl.loop(0, _B_PER_W, step=16)
    def _grp(g):
        vec = idx_v[pl.ds(g, 16)]
        tvec = lax.shift_right_logical(vec, 7)
        cvec = lax.bitwise_and(vec, jnp.full((16,), 127, jnp.int32))

        def fire(l):
            off = pl.multiple_of(tvec[l] * _LANES, _LANES)
            pltpu.async_copy(
                table_t_hbm.at[:, pl.ds(off, _LANES)],
                stages[l % _NBUF],
                sems[l % _NBUF],
            )

        def extract(l):
            pltpu.make_async_copy(
                table_t_hbm.at[:, pl.ds(0, _LANES)],
                stages[l % _NBUF],
                sems[l % _NBUF],
            ).wait()
            col = jnp.full((16,), 1, jnp.int32) * cvec[l]
            pos = jnp.full((16,), 1, jnp.int32) * (g + l)
            for q in range(EMBED_DIM // 16):
                rows = iota16 + (16 * q)
                vals = plsc.load_gather(stages[l % _NBUF], [rows, col])
                plsc.store_scatter(cols_v, [rows, pos], vals)

        for l in range(_NBUF - 1):
            fire(l)
        for l in range(16):
            if l + _NBUF - 1 < 16:
                fire(l + _NBUF - 1)
            extract(l)

    pltpu.sync_copy(cols_v, out_hbm.at[:, pl.ds(base, _B_PER_W)])


def _mm_body(et_ref, w_ref, b_ref, o_ref):
    o_ref[...] = (
        lax.dot_general(
            et_ref[...], w_ref[...],
            (((0,), (1,)), ((), ())),
            preferred_element_type=jnp.float32,
        )
        + b_ref[...]
    )


_M_BLK = 2048


def _tc_linear_t(e_t, fc_w, fc_b2):
    return pl.pallas_call(
        _mm_body,
        grid=(BATCH // _M_BLK,),
        in_specs=[
            pl.BlockSpec((EMBED_DIM, _M_BLK), lambda i: (0, i)),
            pl.BlockSpec((OUTPUT_DIM, EMBED_DIM), lambda i: (0, 0)),
            pl.BlockSpec((1, OUTPUT_DIM), lambda i: (0, 0)),
        ],
        out_specs=pl.BlockSpec((_M_BLK, OUTPUT_DIM), lambda i: (i, 0)),
        out_shape=jax.ShapeDtypeStruct((BATCH, OUTPUT_DIM), jnp.float32),
    )(e_t, fc_w, fc_b2)


def kernel(_x, x, emb_table, fc_w, fc_b):
    idx = x.astype(jnp.int32)
    e_t = _sc_gather_t(emb_table.T, idx)
    return _tc_linear_t(e_t, fc_w, fc_b.reshape(1, OUTPUT_DIM))


# speedup vs baseline: 2.6101x; 1.1266x over previous
import functools

import jax
import jax.numpy as jnp
from jax import lax
from jax.experimental import pallas as pl
from jax.experimental.pallas import tpu as pltpu
from jax.experimental.pallas import tpu_sc as plsc

INPUT_DIM = 1000000
EMBED_DIM = 64
OUTPUT_DIM = 128
BATCH = 16384

_NC = 2
_NS = 16
_NW = _NC * _NS
_B_PER_W = BATCH // _NW
_LANES = 128
_NBUF = 10

_sc_mesh = plsc.VectorSubcoreMesh(core_axis_name="c", subcore_axis_name="s")


@functools.partial(
    pl.kernel,
    mesh=_sc_mesh,
    compiler_params=pltpu.CompilerParams(needs_layout_passes=False),
    out_type=jax.ShapeDtypeStruct((EMBED_DIM, BATCH), jnp.float32),
    scratch_types=[
        pltpu.VMEM((_B_PER_W,), jnp.int32),
        pltpu.VMEM((EMBED_DIM, _B_PER_W), jnp.float32),
        [pltpu.VMEM((EMBED_DIM, _LANES), jnp.float32) for _ in range(_NBUF)],
        [pltpu.SemaphoreType.DMA for _ in range(_NBUF)],
    ],
)
def _sc_gather_t(table_t_hbm, idx_hbm, out_hbm, idx_v, cols_v, stages, sems):
    wid = lax.axis_index("s") * _NC + lax.axis_index("c")
    base = wid * _B_PER_W
    pltpu.sync_copy(idx_hbm.at[pl.ds(base, _B_PER_W)], idx_v)

    iota16 = lax.iota(jnp.int32, 16)

    @pl.loop(0, _B_PER_W, step=16)
    def _grp(g):
        vec = idx_v[pl.ds(g, 16)]
        tvec = lax.shift_right_logical(vec, 7)
        cvec = lax.bitwise_and(vec, jnp.full((16,), 127, jnp.int32))

        def fire(l):
            off = pl.multiple_of(tvec[l] * _LANES, 128)
            pltpu.async_copy(
                table_t_hbm.at[:, pl.ds(off, _LANES)],
                stages[l % _NBUF],
                sems[l % _NBUF],
            )

        def extract(l):
            pltpu.make_async_copy(
                table_t_hbm.at[:, pl.ds(0, _LANES)],
                stages[l % _NBUF],
                sems[l % _NBUF],
            ).wait()
            col = jnp.full((16,), 1, jnp.int32) * cvec[l]
            pos = jnp.full((16,), 1, jnp.int32) * (g + l)
            for q in range(EMBED_DIM // 16):
                rows = iota16 + (16 * q)
                vals = plsc.load_gather(stages[l % _NBUF], [rows, col])
                plsc.store_scatter(cols_v, [rows, pos], vals)

        for l in range(_NBUF - 1):
            fire(l)
        for l in range(16):
            if l + _NBUF - 1 < 16:
                fire(l + _NBUF - 1)
            extract(l)

    pltpu.sync_copy(cols_v, out_hbm.at[:, pl.ds(base, _B_PER_W)])


def _mm_body(et_ref, w_ref, b_ref, o_ref):
    o_ref[...] = (
        lax.dot_general(
            et_ref[...], w_ref[...],
            (((0,), (1,)), ((), ())),
            preferred_element_type=jnp.float32,
        )
        + b_ref[...]
    )


_M_BLK = 2048


def _tc_linear_t(e_t, fc_w, fc_b2):
    return pl.pallas_call(
        _mm_body,
        grid=(BATCH // _M_BLK,),
        in_specs=[
            pl.BlockSpec((EMBED_DIM, _M_BLK), lambda i: (0, i)),
            pl.BlockSpec((OUTPUT_DIM, EMBED_DIM), lambda i: (0, 0)),
            pl.BlockSpec((1, OUTPUT_DIM), lambda i: (0, 0)),
        ],
        out_specs=pl.BlockSpec((_M_BLK, OUTPUT_DIM), lambda i: (i, 0)),
        out_shape=jax.ShapeDtypeStruct((BATCH, OUTPUT_DIM), jnp.float32),
    )(e_t, fc_w, fc_b2)


def kernel(_x, x, emb_table, fc_w, fc_b):
    idx = x.astype(jnp.int32)
    e_t = _sc_gather_t(emb_table.T, idx)
    return _tc_linear_t(e_t, fc_w, fc_b.reshape(1, OUTPUT_DIM))
